# Initial kernel scaffold; baseline (speedup 1.0000x reference)
import jax
import jax.numpy as jnp
from jax.experimental import pallas as pl


def kernel(score_note_dur, phoneme_seq, phoneme_order, emb_word, emb_pos, mix_w1, mix_b1, mix_w2, mix_b2, l0f_wih, l0f_whh, l0f_bih, l0f_bhh, l0b_wih, l0b_whh, l0b_bih, l0b_bhh, l1f_wih, l1f_whh, l1f_bih, l1f_bhh, l1b_wih, l1b_whh, l1b_bih, l1b_bhh, cnn_w1, cnn_b1, cnn_w2, cnn_b2):
    raise NotImplementedError("write your pallas kernel here")


# speedup vs baseline: 7.4400x; 7.4400x over previous
import jax
import jax.numpy as jnp
from jax.experimental import pallas as pl
from jax.experimental.pallas import tpu as pltpu

VOCAB = 100
D = 256
POS = 20
POSD = 10
B = 8
P = 2048
NOTE = 512


def _encoder_body(seq_ref, ord_ref, ew_ref, ep_ref, w1_ref, b1_ref, w2_ref,
                  b2_ref, agg_ref):
    seq = seq_ref[0]
    order = ord_ref[0]

    oh_w = (seq.reshape(P, 1) ==
            jax.lax.broadcasted_iota(jnp.int32, (P, VOCAB), 1)
            ).astype(jnp.float32)
    pe = jnp.dot(oh_w, ew_ref[...], preferred_element_type=jnp.float32)
    oh_p = (order.reshape(P, 1) ==
            jax.lax.broadcasted_iota(jnp.int32, (P, POS), 1)
            ).astype(jnp.float32)
    ppe = jnp.dot(oh_p, ep_ref[...], preferred_element_type=jnp.float32)
    x = jnp.concatenate([pe, ppe], axis=1)

    def conv3(v, w_ref, b_ref):
        cin = v.shape[1]
        vm = jnp.concatenate([jnp.zeros((1, cin), jnp.float32), v[:-1]], axis=0)
        vp = jnp.concatenate([v[1:], jnp.zeros((1, cin), jnp.float32)], axis=0)
        y = (jnp.dot(vm, w_ref[0], preferred_element_type=jnp.float32) +
             jnp.dot(v, w_ref[1], preferred_element_type=jnp.float32) +
             jnp.dot(vp, w_ref[2], preferred_element_type=jnp.float32))
        return y + b_ref[...]

    x = jax.nn.relu(conv3(x, w1_ref, b1_ref))
    x = conv3(x, w2_ref, b2_ref)

    m = (seq > 1) & (jax.lax.broadcasted_iota(jnp.int32, (1, P), 1) < P - 1)
    mi = m.astype(jnp.int32)
    prev = jnp.concatenate([jnp.zeros((1, 1), jnp.int32), mi[:, :-1]], axis=1)
    run_id = mi * (1 - prev)
    k = 1
    while k < P:
        run_id = run_id + jnp.concatenate(
            [jnp.zeros((1, k), jnp.int32), run_id[:, :P - k]], axis=1)
        k *= 2
    run_id = run_id - 1
    seg = jnp.where(m & (run_id < NOTE), run_id, NOTE)

    ohT = (jax.lax.broadcasted_iota(jnp.int32, (NOTE, P), 0) == seg
           ).astype(jnp.float32)
    sums = jnp.dot(ohT, x, preferred_element_type=jnp.float32)
    counts = jnp.sum(ohT, axis=1, keepdims=True)
    agg_ref[0] = sums / jnp.maximum(counts, 1.0)


def _recurrent_body(enc_ref, w0fx_ref, w0fh_ref, b0f_ref,
                    w0bx_ref, w0bh_ref, b0b_ref,
                    w1fx_ref, w1fh_ref, b1f_ref,
                    w1bx_ref, w1bh_ref, b1b_ref,
                    cw1_ref, cb1_ref, cw2_ref, cb2_ref,
                    out_ref, out0_ref, out1_ref):
    F0 = D + 2

    def lstm_cell(gates, c):
        i = jax.nn.sigmoid(gates[:, 0 * D:1 * D])
        f = jax.nn.sigmoid(gates[:, 1 * D:2 * D])
        g = jnp.tanh(gates[:, 2 * D:3 * D])
        o = jax.nn.sigmoid(gates[:, 3 * D:4 * D])
        c = f * c + i * g
        return o * jnp.tanh(c), c

    def layer0_step(t, carry):
        hf, cf, hb, cb = carry
        tb = NOTE - 1 - t
        xf = enc_ref[pl.ds(t, 1)].reshape(B, F0)
        xb = enc_ref[pl.ds(tb, 1)].reshape(B, F0)
        gf = (jnp.dot(xf, w0fx_ref[...], preferred_element_type=jnp.float32) +
              jnp.dot(hf, w0fh_ref[...], preferred_element_type=jnp.float32) +
              b0f_ref[...])
        gb = (jnp.dot(xb, w0bx_ref[...], preferred_element_type=jnp.float32) +
              jnp.dot(hb, w0bh_ref[...], preferred_element_type=jnp.float32) +
              b0b_ref[...])
        hf, cf = lstm_cell(gf, cf)
        hb, cb = lstm_cell(gb, cb)
        out0_ref[pl.ds(t, 1), :, 0:D] = hf.reshape(1, B, D)
        out0_ref[pl.ds(tb, 1), :, D:2 * D] = hb.reshape(1, B, D)
        return hf, cf, hb, cb

    zeros = jnp.zeros((B, D), jnp.float32)
    jax.lax.fori_loop(0, NOTE, layer0_step, (zeros, zeros, zeros, zeros),
                      unroll=False)

    def layer1_step(t, carry):
        hf, cf, hb, cb = carry
        tb = NOTE - 1 - t
        xf = out0_ref[pl.ds(t, 1)].reshape(B, 2 * D)
        xb = out0_ref[pl.ds(tb, 1)].reshape(B, 2 * D)
        gf = (jnp.dot(xf, w1fx_ref[...], preferred_element_type=jnp.float32) +
              jnp.dot(hf, w1fh_ref[...], preferred_element_type=jnp.float32) +
              b1f_ref[...])
        gb = (jnp.dot(xb, w1bx_ref[...], preferred_element_type=jnp.float32) +
              jnp.dot(hb, w1bh_ref[...], preferred_element_type=jnp.float32) +
              b1b_ref[...])
        hf, cf = lstm_cell(gf, cf)
        hb, cb = lstm_cell(gb, cb)
        out1_ref[pl.ds(t, 1), :, 0:D] = hf.reshape(1, B, D)
        out1_ref[pl.ds(tb, 1), :, D:2 * D] = hb.reshape(1, B, D)
        return hf, cf, hb, cb

    jax.lax.fori_loop(0, NOTE, layer1_step, (zeros, zeros, zeros, zeros),
                      unroll=False)

    y = out1_ref[...].reshape(NOTE * B, 2 * D)

    def shift_conv(v, w_ref):
        cin = v.shape[1]
        vm = jnp.concatenate([jnp.zeros((B, cin), jnp.float32), v[:-B]], axis=0)
        vp = jnp.concatenate([v[B:], jnp.zeros((B, cin), jnp.float32)], axis=0)
        return (jnp.dot(vm, w_ref[0], preferred_element_type=jnp.float32) +
                jnp.dot(v, w_ref[1], preferred_element_type=jnp.float32) +
                jnp.dot(vp, w_ref[2], preferred_element_type=jnp.float32))

    y1 = jax.nn.relu(shift_conv(y, cw1_ref) + cb1_ref[...])

    ym = jnp.concatenate([jnp.zeros((B, D), jnp.float32), y1[:-B]], axis=0)
    yp = jnp.concatenate([y1[B:], jnp.zeros((B, D), jnp.float32)], axis=0)
    y2 = (jnp.sum(ym * cw2_ref[0], axis=1, keepdims=True) +
          jnp.sum(y1 * cw2_ref[1], axis=1, keepdims=True) +
          jnp.sum(yp * cw2_ref[2], axis=1, keepdims=True) + cb2_ref[0, 0])
    out_ref[...] = y2.reshape(NOTE, B)


def kernel(score_note_dur, phoneme_seq, phoneme_order, emb_word, emb_pos,
           mix_w1, mix_b1, mix_w2, mix_b2,
           l0f_wih, l0f_whh, l0f_bih, l0f_bhh,
           l0b_wih, l0b_whh, l0b_bih, l0b_bhh,
           l1f_wih, l1f_whh, l1f_bih, l1f_bhh,
           l1b_wih, l1b_whh, l1b_bih, l1b_bhh,
           cnn_w1, cnn_b1, cnn_w2, cnn_b2):
    f32 = jnp.float32
    ew = emb_word.at[0].set(0.0).astype(f32)
    ep = emb_pos.at[0].set(0.0).astype(f32)
    w1 = jnp.transpose(mix_w1, (2, 1, 0)).astype(f32)
    w2 = jnp.transpose(mix_w2, (2, 1, 0)).astype(f32)
    seq3 = phoneme_seq.astype(jnp.int32).reshape(B, 1, P)
    ord3 = phoneme_order.astype(jnp.int32).reshape(B, 1, P)

    agg = pl.pallas_call(
        _encoder_body,
        grid=(B,),
        in_specs=[
            pl.BlockSpec((1, 1, P), lambda b: (b, 0, 0)),
            pl.BlockSpec((1, 1, P), lambda b: (b, 0, 0)),
            pl.BlockSpec((VOCAB, D), lambda b: (0, 0)),
            pl.BlockSpec((POS, POSD), lambda b: (0, 0)),
            pl.BlockSpec((3, D + POSD, D), lambda b: (0, 0, 0)),
            pl.BlockSpec((1, D), lambda b: (0, 0)),
            pl.BlockSpec((3, D, D), lambda b: (0, 0, 0)),
            pl.BlockSpec((1, D), lambda b: (0, 0)),
        ],
        out_specs=pl.BlockSpec((1, NOTE, D), lambda b: (b, 0, 0)),
        out_shape=jax.ShapeDtypeStruct((B, NOTE, D), f32),
        compiler_params=pltpu.CompilerParams(
            dimension_semantics=("arbitrary",)),
    )(seq3, ord3, ew, ep, w1, mix_b1.reshape(1, D).astype(f32),
      w2, mix_b2.reshape(1, D).astype(f32))

    snd = score_note_dur.astype(f32)
    enc = jnp.concatenate(
        [agg, snd[..., None], 1.0 / (snd[..., None] + 1.0)], axis=2)
    enc = jnp.transpose(enc, (1, 0, 2))

    def prep(wih, whh, bih, bhh):
        return (wih.T.astype(f32), whh.T.astype(f32),
                (bih + bhh).reshape(1, 4 * D).astype(f32))

    w0fx, w0fh, b0f = prep(l0f_wih, l0f_whh, l0f_bih, l0f_bhh)
    w0bx, w0bh, b0b = prep(l0b_wih, l0b_whh, l0b_bih, l0b_bhh)
    w1fx, w1fh, b1f = prep(l1f_wih, l1f_whh, l1f_bih, l1f_bhh)
    w1bx, w1bh, b1b = prep(l1b_wih, l1b_whh, l1b_bih, l1b_bhh)
    cw1 = jnp.transpose(cnn_w1, (2, 1, 0)).astype(f32)
    cw2 = jnp.transpose(cnn_w2, (2, 0, 1)).astype(f32).reshape(3, D)
    cw2 = cw2[:, None, :]

    full = lambda shape: pl.BlockSpec(shape, lambda: tuple(0 for _ in shape))
    out = pl.pallas_call(
        _recurrent_body,
        in_specs=[
            full((NOTE, B, D + 2)),
            full((D + 2, 4 * D)), full((D, 4 * D)), full((1, 4 * D)),
            full((D + 2, 4 * D)), full((D, 4 * D)), full((1, 4 * D)),
            full((2 * D, 4 * D)), full((D, 4 * D)), full((1, 4 * D)),
            full((2 * D, 4 * D)), full((D, 4 * D)), full((1, 4 * D)),
            full((3, 2 * D, D)), full((1, D)), full((3, 1, D)),
            full((1, 1)),
        ],
        out_specs=full((NOTE, B)),
        out_shape=jax.ShapeDtypeStruct((NOTE, B), f32),
        scratch_shapes=[
            pltpu.VMEM((NOTE, B, 2 * D), f32),
            pltpu.VMEM((NOTE, B, 2 * D), f32),
        ],
    )(enc,
      w0fx, w0fh, b0f, w0bx, w0bh, b0b,
      w1fx, w1fh, b1f, w1bx, w1bh, b1b,
      cw1, cnn_b1.reshape(1, D).astype(f32), cw2,
      cnn_b2.reshape(1, 1).astype(f32))

    return out.T[..., None]


# speedup vs baseline: 8.4576x; 1.1368x over previous
import jax
import jax.numpy as jnp
from jax.experimental import pallas as pl
from jax.experimental.pallas import tpu as pltpu

VOCAB = 100
D = 256
POS = 20
POSD = 10
B = 8
P = 2048
NOTE = 512


def _encoder_body(seq_ref, ord_ref, ew_ref, ep_ref, w1_ref, b1_ref, w2_ref,
                  b2_ref, agg_ref):
    seq = seq_ref[0]
    order = ord_ref[0]

    oh_w = (seq.reshape(P, 1) ==
            jax.lax.broadcasted_iota(jnp.int32, (P, VOCAB), 1)
            ).astype(jnp.float32)
    pe = jnp.dot(oh_w, ew_ref[...], preferred_element_type=jnp.float32)
    oh_p = (order.reshape(P, 1) ==
            jax.lax.broadcasted_iota(jnp.int32, (P, POS), 1)
            ).astype(jnp.float32)
    ppe = jnp.dot(oh_p, ep_ref[...], preferred_element_type=jnp.float32)
    x = jnp.concatenate([pe, ppe], axis=1)

    def conv3(v, w_ref, b_ref):
        cin = v.shape[1]
        vm = jnp.concatenate([jnp.zeros((1, cin), jnp.float32), v[:-1]], axis=0)
        vp = jnp.concatenate([v[1:], jnp.zeros((1, cin), jnp.float32)], axis=0)
        y = (jnp.dot(vm, w_ref[0], preferred_element_type=jnp.float32) +
             jnp.dot(v, w_ref[1], preferred_element_type=jnp.float32) +
             jnp.dot(vp, w_ref[2], preferred_element_type=jnp.float32))
        return y + b_ref[...]

    x = jax.nn.relu(conv3(x, w1_ref, b1_ref))
    x = conv3(x, w2_ref, b2_ref)

    m = (seq > 1) & (jax.lax.broadcasted_iota(jnp.int32, (1, P), 1) < P - 1)
    mi = m.astype(jnp.int32)
    prev = jnp.concatenate([jnp.zeros((1, 1), jnp.int32), mi[:, :-1]], axis=1)
    run_id = mi * (1 - prev)
    k = 1
    while k < P:
        run_id = run_id + jnp.concatenate(
            [jnp.zeros((1, k), jnp.int32), run_id[:, :P - k]], axis=1)
        k *= 2
    run_id = run_id - 1
    seg = jnp.where(m & (run_id < NOTE), run_id, NOTE)

    ohT = (jax.lax.broadcasted_iota(jnp.int32, (NOTE, P), 0) == seg
           ).astype(jnp.float32)
    sums = jnp.dot(ohT, x, preferred_element_type=jnp.float32)
    counts = jnp.sum(ohT, axis=1, keepdims=True)
    agg_ref[0] = sums / jnp.maximum(counts, 1.0)


def _recurrent_body(enc_ref, w0fx_ref, w0fh_ref, b0f_ref,
                    w0bx_ref, w0bh_ref, b0b_ref,
                    w1fx_ref, w1fh_ref, b1f_ref,
                    w1bx_ref, w1bh_ref, b1b_ref,
                    cw1_ref, cb1_ref, cw2_ref, cb2_ref,
                    out_ref, out0_ref, out1_ref):
    F0 = D + 2

    def lstm_cell(gates, c):
        i = jax.nn.sigmoid(gates[:, 0 * D:1 * D])
        f = jax.nn.sigmoid(gates[:, 1 * D:2 * D])
        g = jnp.tanh(gates[:, 2 * D:3 * D])
        o = jax.nn.sigmoid(gates[:, 3 * D:4 * D])
        c = f * c + i * g
        return o * jnp.tanh(c), c

    def layer0_step(t, carry):
        hf, cf, hb, cb = carry
        tb = NOTE - 1 - t
        xf = enc_ref[pl.ds(t, 1)].reshape(B, F0)
        xb = enc_ref[pl.ds(tb, 1)].reshape(B, F0)
        gf = (jnp.dot(xf, w0fx_ref[...], preferred_element_type=jnp.float32) +
              jnp.dot(hf, w0fh_ref[...], preferred_element_type=jnp.float32) +
              b0f_ref[...])
        gb = (jnp.dot(xb, w0bx_ref[...], preferred_element_type=jnp.float32) +
              jnp.dot(hb, w0bh_ref[...], preferred_element_type=jnp.float32) +
              b0b_ref[...])
        hf, cf = lstm_cell(gf, cf)
        hb, cb = lstm_cell(gb, cb)
        out0_ref[pl.ds(t, 1), :, 0:D] = hf.reshape(1, B, D)
        out0_ref[pl.ds(tb, 1), :, D:2 * D] = hb.reshape(1, B, D)
        return hf, cf, hb, cb

    zeros = jnp.zeros((B, D), jnp.float32)
    jax.lax.fori_loop(0, NOTE, layer0_step, (zeros, zeros, zeros, zeros),
                      unroll=2)

    def layer1_step(t, carry):
        hf, cf, hb, cb = carry
        tb = NOTE - 1 - t
        xf = out0_ref[pl.ds(t, 1)].reshape(B, 2 * D)
        xb = out0_ref[pl.ds(tb, 1)].reshape(B, 2 * D)
        gf = (jnp.dot(xf, w1fx_ref[...], preferred_element_type=jnp.float32) +
              jnp.dot(hf, w1fh_ref[...], preferred_element_type=jnp.float32) +
              b1f_ref[...])
        gb = (jnp.dot(xb, w1bx_ref[...], preferred_element_type=jnp.float32) +
              jnp.dot(hb, w1bh_ref[...], preferred_element_type=jnp.float32) +
              b1b_ref[...])
        hf, cf = lstm_cell(gf, cf)
        hb, cb = lstm_cell(gb, cb)
        out1_ref[pl.ds(t, 1), :, 0:D] = hf.reshape(1, B, D)
        out1_ref[pl.ds(tb, 1), :, D:2 * D] = hb.reshape(1, B, D)
        return hf, cf, hb, cb

    jax.lax.fori_loop(0, NOTE, layer1_step, (zeros, zeros, zeros, zeros),
                      unroll=2)

    y = out1_ref[...].reshape(NOTE * B, 2 * D)

    def shift_conv(v, w_ref):
        cin = v.shape[1]
        vm = jnp.concatenate([jnp.zeros((B, cin), jnp.float32), v[:-B]], axis=0)
        vp = jnp.concatenate([v[B:], jnp.zeros((B, cin), jnp.float32)], axis=0)
        return (jnp.dot(vm, w_ref[0], preferred_element_type=jnp.float32) +
                jnp.dot(v, w_ref[1], preferred_element_type=jnp.float32) +
                jnp.dot(vp, w_ref[2], preferred_element_type=jnp.float32))

    y1 = jax.nn.relu(shift_conv(y, cw1_ref) + cb1_ref[...])

    ym = jnp.concatenate([jnp.zeros((B, D), jnp.float32), y1[:-B]], axis=0)
    yp = jnp.concatenate([y1[B:], jnp.zeros((B, D), jnp.float32)], axis=0)
    y2 = (jnp.sum(ym * cw2_ref[0], axis=1, keepdims=True) +
          jnp.sum(y1 * cw2_ref[1], axis=1, keepdims=True) +
          jnp.sum(yp * cw2_ref[2], axis=1, keepdims=True) + cb2_ref[0, 0])
    out_ref[...] = y2.reshape(NOTE, B)


def kernel(score_note_dur, phoneme_seq, phoneme_order, emb_word, emb_pos,
           mix_w1, mix_b1, mix_w2, mix_b2,
           l0f_wih, l0f_whh, l0f_bih, l0f_bhh,
           l0b_wih, l0b_whh, l0b_bih, l0b_bhh,
           l1f_wih, l1f_whh, l1f_bih, l1f_bhh,
           l1b_wih, l1b_whh, l1b_bih, l1b_bhh,
           cnn_w1, cnn_b1, cnn_w2, cnn_b2):
    f32 = jnp.float32
    ew = emb_word.at[0].set(0.0).astype(f32)
    ep = emb_pos.at[0].set(0.0).astype(f32)
    w1 = jnp.transpose(mix_w1, (2, 1, 0)).astype(f32)
    w2 = jnp.transpose(mix_w2, (2, 1, 0)).astype(f32)
    seq3 = phoneme_seq.astype(jnp.int32).reshape(B, 1, P)
    ord3 = phoneme_order.astype(jnp.int32).reshape(B, 1, P)

    agg = pl.pallas_call(
        _encoder_body,
        grid=(B,),
        in_specs=[
            pl.BlockSpec((1, 1, P), lambda b: (b, 0, 0)),
            pl.BlockSpec((1, 1, P), lambda b: (b, 0, 0)),
            pl.BlockSpec((VOCAB, D), lambda b: (0, 0)),
            pl.BlockSpec((POS, POSD), lambda b: (0, 0)),
            pl.BlockSpec((3, D + POSD, D), lambda b: (0, 0, 0)),
            pl.BlockSpec((1, D), lambda b: (0, 0)),
            pl.BlockSpec((3, D, D), lambda b: (0, 0, 0)),
            pl.BlockSpec((1, D), lambda b: (0, 0)),
        ],
        out_specs=pl.BlockSpec((1, NOTE, D), lambda b: (b, 0, 0)),
        out_shape=jax.ShapeDtypeStruct((B, NOTE, D), f32),
        compiler_params=pltpu.CompilerParams(
            dimension_semantics=("arbitrary",)),
    )(seq3, ord3, ew, ep, w1, mix_b1.reshape(1, D).astype(f32),
      w2, mix_b2.reshape(1, D).astype(f32))

    snd = score_note_dur.astype(f32)
    enc = jnp.concatenate(
        [agg, snd[..., None], 1.0 / (snd[..., None] + 1.0)], axis=2)
    enc = jnp.transpose(enc, (1, 0, 2))

    def prep(wih, whh, bih, bhh):
        return (wih.T.astype(f32), whh.T.astype(f32),
                (bih + bhh).reshape(1, 4 * D).astype(f32))

    w0fx, w0fh, b0f = prep(l0f_wih, l0f_whh, l0f_bih, l0f_bhh)
    w0bx, w0bh, b0b = prep(l0b_wih, l0b_whh, l0b_bih, l0b_bhh)
    w1fx, w1fh, b1f = prep(l1f_wih, l1f_whh, l1f_bih, l1f_bhh)
    w1bx, w1bh, b1b = prep(l1b_wih, l1b_whh, l1b_bih, l1b_bhh)
    cw1 = jnp.transpose(cnn_w1, (2, 1, 0)).astype(f32)
    cw2 = jnp.transpose(cnn_w2, (2, 0, 1)).astype(f32).reshape(3, D)
    cw2 = cw2[:, None, :]

    full = lambda shape: pl.BlockSpec(shape, lambda: tuple(0 for _ in shape))
    out = pl.pallas_call(
        _recurrent_body,
        in_specs=[
            full((NOTE, B, D + 2)),
            full((D + 2, 4 * D)), full((D, 4 * D)), full((1, 4 * D)),
            full((D + 2, 4 * D)), full((D, 4 * D)), full((1, 4 * D)),
            full((2 * D, 4 * D)), full((D, 4 * D)), full((1, 4 * D)),
            full((2 * D, 4 * D)), full((D, 4 * D)), full((1, 4 * D)),
            full((3, 2 * D, D)), full((1, D)), full((3, 1, D)),
            full((1, 1)),
        ],
        out_specs=full((NOTE, B)),
        out_shape=jax.ShapeDtypeStruct((NOTE, B), f32),
        scratch_shapes=[
            pltpu.VMEM((NOTE, B, 2 * D), f32),
            pltpu.VMEM((NOTE, B, 2 * D), f32),
        ],
    )(enc,
      w0fx, w0fh, b0f, w0bx, w0bh, b0b,
      w1fx, w1fh, b1f, w1bx, w1bh, b1b,
      cw1, cnn_b1.reshape(1, D).astype(f32), cw2,
      cnn_b2.reshape(1, 1).astype(f32))

    return out.T[..., None]


# speedup vs baseline: 9.0764x; 1.0732x over previous
import jax
import jax.numpy as jnp
from jax.experimental import pallas as pl
from jax.experimental.pallas import tpu as pltpu

VOCAB = 100
D = 256
POS = 20
POSD = 10
B = 8
P = 2048
NOTE = 512


def _encoder_body(seq_ref, ord_ref, ew_ref, ep_ref, w1_ref, b1_ref, w2_ref,
                  b2_ref, agg_ref):
    seq = seq_ref[0]
    order = ord_ref[0]

    oh_w = (seq.reshape(P, 1) ==
            jax.lax.broadcasted_iota(jnp.int32, (P, VOCAB), 1)
            ).astype(jnp.float32)
    pe = jnp.dot(oh_w, ew_ref[...], preferred_element_type=jnp.float32)
    oh_p = (order.reshape(P, 1) ==
            jax.lax.broadcasted_iota(jnp.int32, (P, POS), 1)
            ).astype(jnp.float32)
    ppe = jnp.dot(oh_p, ep_ref[...], preferred_element_type=jnp.float32)
    x = jnp.concatenate([pe, ppe], axis=1)

    def conv3(v, w_ref, b_ref):
        cin = v.shape[1]
        vm = jnp.concatenate([jnp.zeros((1, cin), jnp.float32), v[:-1]], axis=0)
        vp = jnp.concatenate([v[1:], jnp.zeros((1, cin), jnp.float32)], axis=0)
        y = (jnp.dot(vm, w_ref[0], preferred_element_type=jnp.float32) +
             jnp.dot(v, w_ref[1], preferred_element_type=jnp.float32) +
             jnp.dot(vp, w_ref[2], preferred_element_type=jnp.float32))
        return y + b_ref[...]

    x = jax.nn.relu(conv3(x, w1_ref, b1_ref))
    x = conv3(x, w2_ref, b2_ref)

    m = (seq > 1) & (jax.lax.broadcasted_iota(jnp.int32, (1, P), 1) < P - 1)
    mi = m.astype(jnp.int32)
    prev = jnp.concatenate([jnp.zeros((1, 1), jnp.int32), mi[:, :-1]], axis=1)
    run_id = mi * (1 - prev)
    k = 1
    while k < P:
        run_id = run_id + jnp.concatenate(
            [jnp.zeros((1, k), jnp.int32), run_id[:, :P - k]], axis=1)
        k *= 2
    run_id = run_id - 1
    seg = jnp.where(m & (run_id < NOTE), run_id, NOTE)

    ohT = (jax.lax.broadcasted_iota(jnp.int32, (NOTE, P), 0) == seg
           ).astype(jnp.float32)
    sums = jnp.dot(ohT, x, preferred_element_type=jnp.float32)
    counts = jnp.sum(ohT, axis=1, keepdims=True)
    agg_ref[0] = sums / jnp.maximum(counts, 1.0)


def _recurrent_body(enc_ref, w0fx_ref, w0fh_ref, b0f_ref,
                    w0bx_ref, w0bh_ref, b0b_ref,
                    w1fx_ref, w1fh_ref, b1f_ref,
                    w1bx_ref, w1bh_ref, b1b_ref,
                    cw1_ref, cb1_ref, cw2_ref, cb2_ref,
                    out_ref, out0_ref, out1_ref):
    F0 = D + 2

    def lstm_cell(gates, c):
        i = jax.nn.sigmoid(gates[:, 0 * D:1 * D])
        f = jax.nn.sigmoid(gates[:, 1 * D:2 * D])
        g = jnp.tanh(gates[:, 2 * D:3 * D])
        o = jax.nn.sigmoid(gates[:, 3 * D:4 * D])
        c = f * c + i * g
        return o * jnp.tanh(c), c

    def layer0_step(t, carry):
        hf, cf, hb, cb = carry
        tb = NOTE - 1 - t
        xf = enc_ref[pl.ds(t, 1)].reshape(B, F0)
        xb = enc_ref[pl.ds(tb, 1)].reshape(B, F0)
        gf = (jnp.dot(xf, w0fx_ref[...], preferred_element_type=jnp.float32) +
              jnp.dot(hf, w0fh_ref[...], preferred_element_type=jnp.float32) +
              b0f_ref[...])
        gb = (jnp.dot(xb, w0bx_ref[...], preferred_element_type=jnp.float32) +
              jnp.dot(hb, w0bh_ref[...], preferred_element_type=jnp.float32) +
              b0b_ref[...])
        hf, cf = lstm_cell(gf, cf)
        hb, cb = lstm_cell(gb, cb)
        out0_ref[pl.ds(t, 1), :, 0:D] = hf.reshape(1, B, D)
        out0_ref[pl.ds(tb, 1), :, D:2 * D] = hb.reshape(1, B, D)
        return hf, cf, hb, cb

    zeros = jnp.zeros((B, D), jnp.float32)
    jax.lax.fori_loop(0, NOTE, layer0_step, (zeros, zeros, zeros, zeros),
                      unroll=4)

    def layer1_step(t, carry):
        hf, cf, hb, cb = carry
        tb = NOTE - 1 - t
        xf = out0_ref[pl.ds(t, 1)].reshape(B, 2 * D)
        xb = out0_ref[pl.ds(tb, 1)].reshape(B, 2 * D)
        gf = (jnp.dot(xf, w1fx_ref[...], preferred_element_type=jnp.float32) +
              jnp.dot(hf, w1fh_ref[...], preferred_element_type=jnp.float32) +
              b1f_ref[...])
        gb = (jnp.dot(xb, w1bx_ref[...], preferred_element_type=jnp.float32) +
              jnp.dot(hb, w1bh_ref[...], preferred_element_type=jnp.float32) +
              b1b_ref[...])
        hf, cf = lstm_cell(gf, cf)
        hb, cb = lstm_cell(gb, cb)
        out1_ref[pl.ds(t, 1), :, 0:D] = hf.reshape(1, B, D)
        out1_ref[pl.ds(tb, 1), :, D:2 * D] = hb.reshape(1, B, D)
        return hf, cf, hb, cb

    jax.lax.fori_loop(0, NOTE, layer1_step, (zeros, zeros, zeros, zeros),
                      unroll=4)

    y = out1_ref[...].reshape(NOTE * B, 2 * D)

    def shift_conv(v, w_ref):
        cin = v.shape[1]
        vm = jnp.concatenate([jnp.zeros((B, cin), jnp.float32), v[:-B]], axis=0)
        vp = jnp.concatenate([v[B:], jnp.zeros((B, cin), jnp.float32)], axis=0)
        return (jnp.dot(vm, w_ref[0], preferred_element_type=jnp.float32) +
                jnp.dot(v, w_ref[1], preferred_element_type=jnp.float32) +
                jnp.dot(vp, w_ref[2], preferred_element_type=jnp.float32))

    y1 = jax.nn.relu(shift_conv(y, cw1_ref) + cb1_ref[...])

    ym = jnp.concatenate([jnp.zeros((B, D), jnp.float32), y1[:-B]], axis=0)
    yp = jnp.concatenate([y1[B:], jnp.zeros((B, D), jnp.float32)], axis=0)
    y2 = (jnp.sum(ym * cw2_ref[0], axis=1, keepdims=True) +
          jnp.sum(y1 * cw2_ref[1], axis=1, keepdims=True) +
          jnp.sum(yp * cw2_ref[2], axis=1, keepdims=True) + cb2_ref[0, 0])
    out_ref[...] = y2.reshape(NOTE, B)


def kernel(score_note_dur, phoneme_seq, phoneme_order, emb_word, emb_pos,
           mix_w1, mix_b1, mix_w2, mix_b2,
           l0f_wih, l0f_whh, l0f_bih, l0f_bhh,
           l0b_wih, l0b_whh, l0b_bih, l0b_bhh,
           l1f_wih, l1f_whh, l1f_bih, l1f_bhh,
           l1b_wih, l1b_whh, l1b_bih, l1b_bhh,
           cnn_w1, cnn_b1, cnn_w2, cnn_b2):
    f32 = jnp.float32
    ew = emb_word.at[0].set(0.0).astype(f32)
    ep = emb_pos.at[0].set(0.0).astype(f32)
    w1 = jnp.transpose(mix_w1, (2, 1, 0)).astype(f32)
    w2 = jnp.transpose(mix_w2, (2, 1, 0)).astype(f32)
    seq3 = phoneme_seq.astype(jnp.int32).reshape(B, 1, P)
    ord3 = phoneme_order.astype(jnp.int32).reshape(B, 1, P)

    agg = pl.pallas_call(
        _encoder_body,
        grid=(B,),
        in_specs=[
            pl.BlockSpec((1, 1, P), lambda b: (b, 0, 0)),
            pl.BlockSpec((1, 1, P), lambda b: (b, 0, 0)),
            pl.BlockSpec((VOCAB, D), lambda b: (0, 0)),
            pl.BlockSpec((POS, POSD), lambda b: (0, 0)),
            pl.BlockSpec((3, D + POSD, D), lambda b: (0, 0, 0)),
            pl.BlockSpec((1, D), lambda b: (0, 0)),
            pl.BlockSpec((3, D, D), lambda b: (0, 0, 0)),
            pl.BlockSpec((1, D), lambda b: (0, 0)),
        ],
        out_specs=pl.BlockSpec((1, NOTE, D), lambda b: (b, 0, 0)),
        out_shape=jax.ShapeDtypeStruct((B, NOTE, D), f32),
        compiler_params=pltpu.CompilerParams(
            dimension_semantics=("arbitrary",)),
    )(seq3, ord3, ew, ep, w1, mix_b1.reshape(1, D).astype(f32),
      w2, mix_b2.reshape(1, D).astype(f32))

    snd = score_note_dur.astype(f32)
    enc = jnp.concatenate(
        [agg, snd[..., None], 1.0 / (snd[..., None] + 1.0)], axis=2)
    enc = jnp.transpose(enc, (1, 0, 2))

    def prep(wih, whh, bih, bhh):
        return (wih.T.astype(f32), whh.T.astype(f32),
                (bih + bhh).reshape(1, 4 * D).astype(f32))

    w0fx, w0fh, b0f = prep(l0f_wih, l0f_whh, l0f_bih, l0f_bhh)
    w0bx, w0bh, b0b = prep(l0b_wih, l0b_whh, l0b_bih, l0b_bhh)
    w1fx, w1fh, b1f = prep(l1f_wih, l1f_whh, l1f_bih, l1f_bhh)
    w1bx, w1bh, b1b = prep(l1b_wih, l1b_whh, l1b_bih, l1b_bhh)
    cw1 = jnp.transpose(cnn_w1, (2, 1, 0)).astype(f32)
    cw2 = jnp.transpose(cnn_w2, (2, 0, 1)).astype(f32).reshape(3, D)
    cw2 = cw2[:, None, :]

    full = lambda shape: pl.BlockSpec(shape, lambda: tuple(0 for _ in shape))
    out = pl.pallas_call(
        _recurrent_body,
        in_specs=[
            full((NOTE, B, D + 2)),
            full((D + 2, 4 * D)), full((D, 4 * D)), full((1, 4 * D)),
            full((D + 2, 4 * D)), full((D, 4 * D)), full((1, 4 * D)),
            full((2 * D, 4 * D)), full((D, 4 * D)), full((1, 4 * D)),
            full((2 * D, 4 * D)), full((D, 4 * D)), full((1, 4 * D)),
            full((3, 2 * D, D)), full((1, D)), full((3, 1, D)),
            full((1, 1)),
        ],
        out_specs=full((NOTE, B)),
        out_shape=jax.ShapeDtypeStruct((NOTE, B), f32),
        scratch_shapes=[
            pltpu.VMEM((NOTE, B, 2 * D), f32),
            pltpu.VMEM((NOTE, B, 2 * D), f32),
        ],
    )(enc,
      w0fx, w0fh, b0f, w0bx, w0bh, b0b,
      w1fx, w1fh, b1f, w1bx, w1bh, b1b,
      cw1, cnn_b1.reshape(1, D).astype(f32), cw2,
      cnn_b2.reshape(1, 1).astype(f32))

    return out.T[..., None]
